# baseline (speedup 1.0000x reference)
import jax
import jax.numpy as jnp
from jax import lax
from jax.experimental import pallas as pl
from jax.experimental.pallas import tpu as pltpu
from jax.experimental.pallas import tpu_sc as plsc

B = 1024
V = 100000
NCAND = 50
NCAND_PAD = 64
L = 16

_info = plsc.get_sparse_core_info()
_NC, _NS = _info.num_cores, _info.num_subcores
NW = _NC * _NS

N_SLABS = 4
N_STRIPES = 8
SLAB_V = V // N_SLABS
STRIPE_B = B // N_STRIPES
CV = 200
NCH = SLAB_V // CV
CHW = CV * STRIPE_B
STRIPE_IDX = STRIPE_B * NCAND_PAD
LST_CAP = STRIPE_IDX + L
SENTINEL = 2**30
NBUF = 3
XROWS = B // NW


def _zero_buf(buf):
    zeros16 = jnp.zeros((L,), jnp.float32)

    def zb(k, carry):
        buf[k >> 3, pl.ds((k & 7) * L, L)] = zeros16
        return carry

    lax.fori_loop(0, CV * (STRIPE_B // L), zb, 0, unroll=8)


def _sc_body(idx_hbm, xt_hbm, x0_hbm, x1_hbm, outT_hbm,
             idx_stage, lst, hits0, hits1, hits2, buf0, buf1, buf2, xbuf,
             sem0, sem1, sem2):
    c = lax.axis_index("c")
    s = lax.axis_index("s")
    wid = s * _NC + c
    stripe = wid % N_STRIPES
    slab = wid // N_STRIPES
    vlo = slab * SLAB_V

    xbase = wid * XROWS
    pltpu.sync_copy(xt_hbm.at[0, pl.ds(xbase, XROWS)], xbuf)
    pltpu.sync_copy(xbuf, x0_hbm.at[pl.ds(xbase, XROWS)])
    pltpu.sync_copy(xt_hbm.at[1, pl.ds(xbase, XROWS)], xbuf)
    pltpu.sync_copy(xbuf, x1_hbm.at[pl.ds(xbase, XROWS)])

    pltpu.sync_copy(idx_hbm.at[pl.ds(stripe * STRIPE_IDX, STRIPE_IDX)],
                    idx_stage)

    iota16 = lax.iota(jnp.int32, L)
    ones16 = jnp.ones((L,), jnp.float32)
    zeros16 = jnp.zeros((L,), jnp.float32)
    sent16 = jnp.full((L,), SENTINEL, jnp.int32)

    _zero_buf(buf0)
    _zero_buf(buf1)
    _zero_buf(buf2)

    def fbody(j, cnt):
        iv = idx_stage[pl.ds(j * L, L)]
        v = iv - 1
        m = (iv > 0) & (v >= vlo) & (v < vlo + SLAB_V)
        b_local = (j * L + iota16) >> 6
        off = (v - vlo) * STRIPE_B + b_local
        plsc.store_compressed(lst.at[pl.ds(cnt, L)], off, mask=m)
        return cnt + jnp.sum(m.astype(jnp.int32))

    cnt = lax.fori_loop(0, STRIPE_IDX // L, fbody, jnp.int32(0))
    lst[pl.ds(cnt, L)] = sent16
    n_iter = (cnt + (L - 1)) >> 4

    dst_b = stripe * STRIPE_B

    def chunk_dst(ci):
        return outT_hbm.at[pl.ds(vlo + ci * CV, CV), pl.ds(dst_b, STRIPE_B)]

    def do_chunk(ci, buf, hits, sem, h_in):
        @pl.when(ci >= NBUF)
        def _():
            pltpu.make_async_copy(buf, chunk_dst(ci - NBUF), sem).wait()

        def ub(j, carry):
            e = hits[pl.ds(j * L, L)]
            plsc.store_scatter(buf, [e >> 7, e & (STRIPE_B - 1)], zeros16,
                               mask=e < CHW)
            return carry

        lax.fori_loop(0, (h_in + (L - 1)) >> 4, ub, 0)

        lo = ci * CHW

        def sb(j, hcnt):
            e = lst[pl.ds(j * L, L)]
            rel = e - lo
            m = (rel >= 0) & (rel < CHW)
            plsc.store_scatter(buf, [rel >> 7, rel & (STRIPE_B - 1)], ones16,
                               mask=m)
            plsc.store_compressed(hits.at[pl.ds(hcnt, L)], rel, mask=m)
            return hcnt + jnp.sum(m.astype(jnp.int32))

        h_out = lax.fori_loop(0, n_iter, sb, jnp.int32(0))
        hits[pl.ds(h_out, L)] = sent16
        pltpu.make_async_copy(buf, chunk_dst(ci), sem).start()
        return h_out

    bufs = (buf0, buf1, buf2)
    hitss = (hits0, hits1, hits2)
    sems = (sem0, sem1, sem2)

    def rnd(i, carry):
        hs = list(carry)
        for k in range(NBUF):
            hs[k] = do_chunk(NBUF * i + k, bufs[k], hitss[k], sems[k], hs[k])
        return tuple(hs)

    hs = lax.fori_loop(0, NCH // NBUF, rnd,
                       (jnp.int32(0),) * NBUF)
    hs = list(hs)
    for k in range(NCH % NBUF):
        ci = jnp.int32((NCH // NBUF) * NBUF + k)
        hs[k] = do_chunk(ci, bufs[k], hitss[k], sems[k], hs[k])
    for k in range(NBUF):
        last_ci = NCH - NBUF + ((k - NCH) % NBUF)
        pltpu.make_async_copy(bufs[k], chunk_dst(jnp.int32(last_ci)),
                              sems[k]).wait()


@jax.jit
def _teacher_force(x, outmask, t):
    idx = lax.dynamic_index_in_dim(outmask, t, 1, keepdims=False)
    idx = idx[:, 2:]
    idx = jnp.concatenate(
        [idx, jnp.zeros((B, NCAND_PAD - NCAND), jnp.int32)], axis=1
    )
    idx_flat = idx.reshape(-1)
    x_t = lax.dynamic_index_in_dim(x, t, 2, keepdims=False)

    mesh = plsc.VectorSubcoreMesh(core_axis_name="c", subcore_axis_name="s")
    k = pl.kernel(
        _sc_body,
        mesh=mesh,
        out_type=(
            jax.ShapeDtypeStruct((B, 128), jnp.float32),
            jax.ShapeDtypeStruct((B, 128), jnp.float32),
            jax.ShapeDtypeStruct((V, B), jnp.float32),
        ),
        scratch_types=[
            pltpu.VMEM((STRIPE_IDX,), jnp.int32),
            pltpu.VMEM((LST_CAP,), jnp.int32),
            pltpu.VMEM((LST_CAP,), jnp.int32),
            pltpu.VMEM((LST_CAP,), jnp.int32),
            pltpu.VMEM((LST_CAP,), jnp.int32),
            pltpu.VMEM((CV, STRIPE_B), jnp.float32),
            pltpu.VMEM((CV, STRIPE_B), jnp.float32),
            pltpu.VMEM((CV, STRIPE_B), jnp.float32),
            pltpu.VMEM((XROWS, 128), jnp.float32),
            pltpu.SemaphoreType.DMA,
            pltpu.SemaphoreType.DMA,
            pltpu.SemaphoreType.DMA,
        ],
        compiler_params=pltpu.CompilerParams(needs_layout_passes=False),
    )
    x0, x1, outT = k(idx_flat, x_t)
    return x0, x1, outT.T


def kernel(x, outmask, t):
    return _teacher_force(x, outmask, t)


# speedup vs baseline: 1.0044x; 1.0044x over previous
import jax
import jax.numpy as jnp
from jax import lax
from jax.experimental import pallas as pl
from jax.experimental.pallas import tpu as pltpu
from jax.experimental.pallas import tpu_sc as plsc

B = 1024
V = 100000
NCAND = 50
NCAND_PAD = 64
L = 16

_info = plsc.get_sparse_core_info()
_NC, _NS = _info.num_cores, _info.num_subcores
NW = _NC * _NS

N_SLABS = 4
N_STRIPES = 8
SLAB_V = V // N_SLABS
STRIPE_B = B // N_STRIPES
CV = 200
NCH = SLAB_V // CV
CHW = CV * STRIPE_B
STRIPE_IDX = STRIPE_B * NCAND_PAD
LST_CAP = STRIPE_IDX + L
SENTINEL = 2**30
NBUF = 3
XROWS = B // NW


def _zero_buf(buf):
    zeros16 = jnp.zeros((L,), jnp.float32)

    def zb(k, carry):
        buf[k >> 3, pl.ds((k & 7) * L, L)] = zeros16
        return carry

    lax.fori_loop(0, CV * (STRIPE_B // L), zb, 0, unroll=8)


def _sc_body(idx_hbm, xt_hbm, x0_hbm, x1_hbm, outT_hbm,
             idx_stage, lst, hits0, hits1, hits2, buf0, buf1, buf2, xbuf,
             sem0, sem1, sem2):
    c = lax.axis_index("c")
    s = lax.axis_index("s")
    wid = s * _NC + c
    stripe = wid % N_STRIPES
    slab = wid // N_STRIPES
    vlo = slab * SLAB_V

    pltpu.sync_copy(idx_hbm.at[pl.ds(stripe * STRIPE_IDX, STRIPE_IDX)],
                    idx_stage)

    iota16 = lax.iota(jnp.int32, L)
    ones16 = jnp.ones((L,), jnp.float32)
    zeros16 = jnp.zeros((L,), jnp.float32)
    sent16 = jnp.full((L,), SENTINEL, jnp.int32)

    _zero_buf(buf0)

    def fbody(j, cnt):
        iv = idx_stage[pl.ds(j * L, L)]
        v = iv - 1
        m = (iv > 0) & (v >= vlo) & (v < vlo + SLAB_V)
        b_local = (j * L + iota16) >> 6
        off = (v - vlo) * STRIPE_B + b_local
        plsc.store_compressed(lst.at[pl.ds(cnt, L)], off, mask=m)
        return cnt + jnp.sum(m.astype(jnp.int32))

    cnt = lax.fori_loop(0, STRIPE_IDX // L, fbody, jnp.int32(0))
    lst[pl.ds(cnt, L)] = sent16
    n_iter = (cnt + (L - 1)) >> 4

    dst_b = stripe * STRIPE_B

    def chunk_dst(ci):
        return outT_hbm.at[pl.ds(vlo + ci * CV, CV), pl.ds(dst_b, STRIPE_B)]

    def do_chunk(ci, buf, hits, sem, h_in):
        @pl.when(ci >= NBUF)
        def _():
            pltpu.make_async_copy(buf, chunk_dst(ci - NBUF), sem).wait()

        def ub(j, carry):
            e = hits[pl.ds(j * L, L)]
            plsc.store_scatter(buf, [e >> 7, e & (STRIPE_B - 1)], zeros16,
                               mask=e < CHW)
            return carry

        lax.fori_loop(0, (h_in + (L - 1)) >> 4, ub, 0)

        lo = ci * CHW

        def sb(j, hcnt):
            e = lst[pl.ds(j * L, L)]
            rel = e - lo
            m = (rel >= 0) & (rel < CHW)
            plsc.store_scatter(buf, [rel >> 7, rel & (STRIPE_B - 1)], ones16,
                               mask=m)
            plsc.store_compressed(hits.at[pl.ds(hcnt, L)], rel, mask=m)
            return hcnt + jnp.sum(m.astype(jnp.int32))

        h_out = lax.fori_loop(0, n_iter, sb, jnp.int32(0))
        hits[pl.ds(h_out, L)] = sent16
        pltpu.make_async_copy(buf, chunk_dst(ci), sem).start()
        return h_out

    bufs = (buf0, buf1, buf2)
    hitss = (hits0, hits1, hits2)
    sems = (sem0, sem1, sem2)

    def rnd(i, carry):
        hs = list(carry)
        for k in range(NBUF):
            hs[k] = do_chunk(NBUF * i + k, bufs[k], hitss[k], sems[k], hs[k])
        return tuple(hs)

    hs = [jnp.int32(0)] * NBUF
    hs[0] = do_chunk(jnp.int32(0), buf0, hits0, sem0, hs[0])
    _zero_buf(buf1)
    hs[1] = do_chunk(jnp.int32(1), buf1, hits1, sem1, hs[1])
    _zero_buf(buf2)
    hs[2] = do_chunk(jnp.int32(2), buf2, hits2, sem2, hs[2])

    hs = lax.fori_loop(1, NCH // NBUF, rnd, tuple(hs))
    hs = list(hs)
    for k in range(NCH % NBUF):
        ci = jnp.int32((NCH // NBUF) * NBUF + k)
        hs[k] = do_chunk(ci, bufs[k], hitss[k], sems[k], hs[k])
    xbase = wid * XROWS
    pltpu.sync_copy(xt_hbm.at[0, pl.ds(xbase, XROWS)], xbuf)
    pltpu.sync_copy(xbuf, x0_hbm.at[pl.ds(xbase, XROWS)])
    pltpu.sync_copy(xt_hbm.at[1, pl.ds(xbase, XROWS)], xbuf)
    pltpu.sync_copy(xbuf, x1_hbm.at[pl.ds(xbase, XROWS)])

    for k in range(NBUF):
        last_ci = NCH - NBUF + ((k - NCH) % NBUF)
        pltpu.make_async_copy(bufs[k], chunk_dst(jnp.int32(last_ci)),
                              sems[k]).wait()


@jax.jit
def _teacher_force(x, outmask, t):
    idx = lax.dynamic_index_in_dim(outmask, t, 1, keepdims=False)
    idx = idx[:, 2:]
    idx = jnp.concatenate(
        [idx, jnp.zeros((B, NCAND_PAD - NCAND), jnp.int32)], axis=1
    )
    idx_flat = idx.reshape(-1)
    x_t = lax.dynamic_index_in_dim(x, t, 2, keepdims=False)

    mesh = plsc.VectorSubcoreMesh(core_axis_name="c", subcore_axis_name="s")
    k = pl.kernel(
        _sc_body,
        mesh=mesh,
        out_type=(
            jax.ShapeDtypeStruct((B, 128), jnp.float32),
            jax.ShapeDtypeStruct((B, 128), jnp.float32),
            jax.ShapeDtypeStruct((V, B), jnp.float32),
        ),
        scratch_types=[
            pltpu.VMEM((STRIPE_IDX,), jnp.int32),
            pltpu.VMEM((LST_CAP,), jnp.int32),
            pltpu.VMEM((LST_CAP,), jnp.int32),
            pltpu.VMEM((LST_CAP,), jnp.int32),
            pltpu.VMEM((LST_CAP,), jnp.int32),
            pltpu.VMEM((CV, STRIPE_B), jnp.float32),
            pltpu.VMEM((CV, STRIPE_B), jnp.float32),
            pltpu.VMEM((CV, STRIPE_B), jnp.float32),
            pltpu.VMEM((XROWS, 128), jnp.float32),
            pltpu.SemaphoreType.DMA,
            pltpu.SemaphoreType.DMA,
            pltpu.SemaphoreType.DMA,
        ],
        compiler_params=pltpu.CompilerParams(needs_layout_passes=False),
    )
    x0, x1, outT = k(idx_flat, x_t)
    return x0, x1, outT.T


def kernel(x, outmask, t):
    return _teacher_force(x, outmask, t)


# speedup vs baseline: 1.0377x; 1.0332x over previous
import jax
import jax.numpy as jnp
from jax import lax
from jax.experimental import pallas as pl
from jax.experimental.pallas import tpu as pltpu
from jax.experimental.pallas import tpu_sc as plsc

B = 1024
V = 100000
NCAND = 50
NCAND_PAD = 64
L = 16

_info = plsc.get_sparse_core_info()
_NC, _NS = _info.num_cores, _info.num_subcores
NW = _NC * _NS

N_SLABS = 4
N_STRIPES = 8
SLAB_V = V // N_SLABS
STRIPE_B = B // N_STRIPES
CV = 200
NCH = SLAB_V // CV
CHW = CV * STRIPE_B
STRIPE_IDX = STRIPE_B * NCAND_PAD
LST_CAP = STRIPE_IDX + L
SENTINEL = 2**30
NBUF = 3
XROWS = B // NW


def _zero_buf(buf):
    zeros16 = jnp.zeros((L,), jnp.float32)

    def zb(k, carry):
        buf[k >> 3, pl.ds((k & 7) * L, L)] = zeros16
        return carry

    lax.fori_loop(0, CV * (STRIPE_B // L), zb, 0, unroll=8)


def _sc_body(idx_hbm, outT_hbm,
             idx_stage, lst, hits0, hits1, hits2, buf0, buf1, buf2,
             sem0, sem1, sem2):
    c = lax.axis_index("c")
    s = lax.axis_index("s")
    wid = s * _NC + c
    stripe = wid % N_STRIPES
    slab = wid // N_STRIPES
    vlo = slab * SLAB_V

    pltpu.sync_copy(idx_hbm.at[pl.ds(stripe * STRIPE_IDX, STRIPE_IDX)],
                    idx_stage)

    iota16 = lax.iota(jnp.int32, L)
    ones16 = jnp.ones((L,), jnp.float32)
    zeros16 = jnp.zeros((L,), jnp.float32)
    sent16 = jnp.full((L,), SENTINEL, jnp.int32)

    _zero_buf(buf0)

    def fbody(j, cnt):
        iv = idx_stage[pl.ds(j * L, L)]
        v = iv - 1
        m = (iv > 0) & (v >= vlo) & (v < vlo + SLAB_V)
        b_local = (j * L + iota16) >> 6
        off = (v - vlo) * STRIPE_B + b_local
        plsc.store_compressed(lst.at[pl.ds(cnt, L)], off, mask=m)
        return cnt + jnp.sum(m.astype(jnp.int32))

    cnt = lax.fori_loop(0, STRIPE_IDX // L, fbody, jnp.int32(0))
    lst[pl.ds(cnt, L)] = sent16
    n_iter = (cnt + (L - 1)) >> 4

    dst_b = stripe * STRIPE_B

    def chunk_dst(ci):
        return outT_hbm.at[pl.ds(vlo + ci * CV, CV), pl.ds(dst_b, STRIPE_B)]

    def do_chunk(ci, buf, hits, sem, h_in):
        @pl.when(ci >= NBUF)
        def _():
            pltpu.make_async_copy(buf, chunk_dst(ci - NBUF), sem).wait()

        def ub(j, carry):
            e = hits[pl.ds(j * L, L)]
            plsc.store_scatter(buf, [e >> 7, e & (STRIPE_B - 1)], zeros16,
                               mask=e < CHW)
            return carry

        lax.fori_loop(0, (h_in + (L - 1)) >> 4, ub, 0)

        lo = ci * CHW

        def sb(j, hcnt):
            e = lst[pl.ds(j * L, L)]
            rel = e - lo
            m = (rel >= 0) & (rel < CHW)
            plsc.store_scatter(buf, [rel >> 7, rel & (STRIPE_B - 1)], ones16,
                               mask=m)
            plsc.store_compressed(hits.at[pl.ds(hcnt, L)], rel, mask=m)
            return hcnt + jnp.sum(m.astype(jnp.int32))

        h_out = lax.fori_loop(0, n_iter, sb, jnp.int32(0))
        hits[pl.ds(h_out, L)] = sent16
        pltpu.make_async_copy(buf, chunk_dst(ci), sem).start()
        return h_out

    bufs = (buf0, buf1, buf2)
    hitss = (hits0, hits1, hits2)
    sems = (sem0, sem1, sem2)

    def rnd(i, carry):
        hs = list(carry)
        for k in range(NBUF):
            hs[k] = do_chunk(NBUF * i + k, bufs[k], hitss[k], sems[k], hs[k])
        return tuple(hs)

    hs = [jnp.int32(0)] * NBUF
    hs[0] = do_chunk(jnp.int32(0), buf0, hits0, sem0, hs[0])
    _zero_buf(buf1)
    hs[1] = do_chunk(jnp.int32(1), buf1, hits1, sem1, hs[1])
    _zero_buf(buf2)
    hs[2] = do_chunk(jnp.int32(2), buf2, hits2, sem2, hs[2])

    hs = lax.fori_loop(1, NCH // NBUF, rnd, tuple(hs))
    hs = list(hs)
    for k in range(NCH % NBUF):
        ci = jnp.int32((NCH // NBUF) * NBUF + k)
        hs[k] = do_chunk(ci, bufs[k], hitss[k], sems[k], hs[k])
    for k in range(NBUF):
        last_ci = NCH - NBUF + ((k - NCH) % NBUF)
        pltpu.make_async_copy(bufs[k], chunk_dst(jnp.int32(last_ci)),
                              sems[k]).wait()


@jax.jit
def _teacher_force(x, outmask, t):
    idx = lax.dynamic_index_in_dim(outmask, t, 1, keepdims=False)
    idx = idx[:, 2:]
    idx = jnp.concatenate(
        [idx, jnp.zeros((B, NCAND_PAD - NCAND), jnp.int32)], axis=1
    )
    idx_flat = idx.reshape(-1)
    x_t = lax.dynamic_index_in_dim(x, t, 2, keepdims=False)

    mesh = plsc.VectorSubcoreMesh(core_axis_name="c", subcore_axis_name="s")
    k = pl.kernel(
        _sc_body,
        mesh=mesh,
        out_type=jax.ShapeDtypeStruct((V, B), jnp.float32),
        scratch_types=[
            pltpu.VMEM((STRIPE_IDX,), jnp.int32),
            pltpu.VMEM((LST_CAP,), jnp.int32),
            pltpu.VMEM((LST_CAP,), jnp.int32),
            pltpu.VMEM((LST_CAP,), jnp.int32),
            pltpu.VMEM((LST_CAP,), jnp.int32),
            pltpu.VMEM((CV, STRIPE_B), jnp.float32),
            pltpu.VMEM((CV, STRIPE_B), jnp.float32),
            pltpu.VMEM((CV, STRIPE_B), jnp.float32),
            pltpu.SemaphoreType.DMA,
            pltpu.SemaphoreType.DMA,
            pltpu.SemaphoreType.DMA,
        ],
        compiler_params=pltpu.CompilerParams(needs_layout_passes=False),
    )
    outT = k(idx_flat)

    def _xcopy(xt_ref, o0_ref, o1_ref):
        o0_ref[...] = xt_ref[0]
        o1_ref[...] = xt_ref[1]

    x0, x1 = pl.pallas_call(
        _xcopy,
        out_shape=(
            jax.ShapeDtypeStruct((B, 128), jnp.float32),
            jax.ShapeDtypeStruct((B, 128), jnp.float32),
        ),
    )(x_t)
    return x0, x1, outT.T


def kernel(x, outmask, t):
    return _teacher_force(x, outmask, t)
